# baseline (speedup 1.0000x reference)
import jax
import jax.numpy as jnp
from jax import lax
from jax.experimental import pallas as pl
from jax.experimental.pallas import tpu as pltpu
from jax.experimental.pallas import tpu_sc as plsc

_NE = 1000000
_D = 64
_B = 16384

_NC = 2
_NS = 16
_NW = _NC * _NS
_EPW = 31232
_EC = 4096
_NCHUNK = 8
_TAILBASE = 999936
_ECLAMP = _TAILBASE - _EC
_MSLOT = 1024
_CSLOT = 256
_NREG = _NCHUNK + 1
_DUMMY = _B


def _lane(v16, s):
    return lax.squeeze(lax.slice(v16, (s,), (s + 1,)), (0,))


def _scan_body(tT, table, idx_hbm, inter_out, posq_out, idxbuf, pbuf, ment,
               mpos, cent, cpos, outchunk, sem):
    wid = lax.axis_index("s") * _NC + lax.axis_index("c")
    ws = wid * _EPW
    we = jnp.where(wid == _NW - 1, _NE, ws + _EPW)
    iota = lax.iota(jnp.int32, 16)
    k16 = jnp.full((16,), _DUMMY, jnp.int32)

    def pcpos(i, c):
        cpos[pl.ds(i * 16, 16)] = k16
        return c

    neg = jnp.full((16,), -1, jnp.int32)

    def pment(i, c):
        ment[pl.ds(i * 16, 16)] = neg
        return c

    lax.fori_loop(0, _MSLOT // 16, pment, 0)

    mc = jnp.int32(0)
    for piece in range(4):
        pltpu.sync_copy(idx_hbm.at[pl.ds(piece * 4096, 4096)], idxbuf)

        def mbody(g, mc, piece=piece):
            v = idxbuf[pl.ds(g * 16, 16)]
            p16 = piece * 4096 + g * 16 + iota
            m = (v >= ws) & (v < we)
            mcc = jnp.minimum(mc, _MSLOT - 16)
            plsc.store_compressed(ment.at[pl.ds(mcc, 16)], v, mask=m)
            plsc.store_compressed(mpos.at[pl.ds(mcc, 16)], p16, mask=m)
            return mc + _lane(plsc.all_reduce_population_count(m), 0)

        mc = lax.fori_loop(0, 256, mbody, mc)

    def chunk_e0(k):
        return pl.multiple_of(jnp.minimum(ws + k * _EC, _ECLAMP), 128)

    def fire(k, p, b):
        e0 = chunk_e0(k)
        pltpu.async_copy(
            tT.at[pl.ds(p * 8, 8), pl.ds(e0, _EC)], pbuf.at[b], sem
        )

    fire(0, 0, 0)
    for k in range(_NCHUNK):
        e0 = chunk_e0(k)
        lo = ws + k * _EC
        hi = jnp.minimum(jnp.minimum(lo + _EC, we), e0 + _EC)

        lax.fori_loop(0, _CSLOT // 16, pcpos, 0)

        def cbody(g, cc):
            e16 = ment[pl.ds(g * 16, 16)]
            q16 = mpos[pl.ds(g * 16, 16)]
            m = (e16 >= lo) & (e16 < hi)
            ccc = jnp.minimum(cc, _CSLOT - 16)
            plsc.store_compressed(cent.at[pl.ds(ccc, 16)], e16, mask=m)
            plsc.store_compressed(cpos.at[pl.ds(ccc, 16)], q16, mask=m)
            return cc + _lane(plsc.all_reduce_population_count(m), 0)

        cc = lax.fori_loop(0, _MSLOT // 16, cbody, jnp.int32(0))

        for p in range(8):
            b = (k * 8 + p) % 2
            pltpu.make_async_copy(
                tT.at[pl.ds(0, 8), pl.ds(0, _EC)], pbuf.at[b], sem
            ).wait()
            if p < 7:
                fire(k, p + 1, 1 - b)
            elif k + 1 < _NCHUNK:
                fire(k + 1, 0, 1 - b)
            def ebody(j, c, b=b, p=p, cc=cc, e0=e0):
                slot = j * 16 + iota
                sm = slot < cc
                colv = jnp.clip(cent[pl.ds(j * 16, 16)] - e0, 0, _EC - 1)
                for f in range(8):
                    fv = jnp.full((16,), f, jnp.int32)
                    val = plsc.load_gather(pbuf.at[b], [fv, colv], mask=sm)
                    cv = jnp.full((16,), p * 8 + f, jnp.int32)
                    plsc.store_scatter(outchunk, [slot, cv], val, mask=sm)
                return c

            lax.fori_loop(0, _CSLOT // 16, ebody, 0)

        pltpu.sync_copy(outchunk, inter_out.at[wid * _NREG + k])
        pltpu.sync_copy(cpos, posq_out.at[pl.ds((wid * _NREG + k) * _CSLOT, _CSLOT)])

    lax.fori_loop(0, _CSLOT // 16, pcpos, 0)

    def tbody(g, cc):
        e16 = ment[pl.ds(g * 16, 16)]
        q16 = mpos[pl.ds(g * 16, 16)]
        m = e16 >= _TAILBASE
        ccc = jnp.minimum(cc, _CSLOT - 16)
        plsc.store_compressed(cent.at[pl.ds(ccc, 16)], e16, mask=m)
        plsc.store_compressed(cpos.at[pl.ds(ccc, 16)], q16, mask=m)
        return cc + _lane(plsc.all_reduce_population_count(m), 0)

    cc_t = lax.fori_loop(0, _MSLOT // 16, tbody, jnp.int32(0))
    cent0 = cent[pl.ds(0, 16)]
    for s in range(16):
        es = _lane(cent0, s)

        @pl.when(s < cc_t)
        def _():
            pltpu.sync_copy(
                table.at[pl.ds(es, 1)], outchunk.at[pl.ds(s, 1)]
            )

    pltpu.sync_copy(outchunk, inter_out.at[wid * _NREG + _NCHUNK])
    pltpu.sync_copy(cpos, posq_out.at[pl.ds((wid * _NREG + _NCHUNK) * _CSLOT, _CSLOT)])


def _scatter_body(inter, posq, out_hbm, rows_v, pos_v, sem):
    wid = lax.axis_index("s") * _NC + lax.axis_index("c")
    for r in range(_NREG):
        pltpu.sync_copy(inter.at[wid * _NREG + r], rows_v)
        pltpu.sync_copy(posq.at[pl.ds((wid * _NREG + r) * 2, 2)], pos_v)
        copies = [
            pltpu.async_copy(
                rows_v.at[pl.ds(j * 128, 128)], out_hbm.at[pos_v.at[j]], sem
            )
            for j in range(2)
        ]
        for c in copies:
            c.wait()


def kernel(entities, table):
    idx = entities.astype(jnp.int32)
    tT = table.T
    mesh = plsc.VectorSubcoreMesh(core_axis_name="c", subcore_axis_name="s")
    inter, posq = pl.kernel(
        _scan_body,
        out_type=(
            jax.ShapeDtypeStruct((_NW * _NREG, _CSLOT, _D), jnp.float32),
            jax.ShapeDtypeStruct((_NW * _NREG * _CSLOT,), jnp.int32),
        ),
        mesh=mesh,
        scratch_types=[
            pltpu.VMEM((4096,), jnp.int32),
            pltpu.VMEM((2, 8, _EC), jnp.float32),
            pltpu.VMEM((_MSLOT,), jnp.int32),
            pltpu.VMEM((_MSLOT,), jnp.int32),
            pltpu.VMEM((_CSLOT,), jnp.int32),
            pltpu.VMEM((_CSLOT,), jnp.int32),
            pltpu.VMEM((_CSLOT, _D), jnp.float32),
            pltpu.SemaphoreType.DMA,
        ],
        compiler_params=pltpu.CompilerParams(
            use_tc_tiling_on_sc=True, needs_layout_passes=False
        ),
    )(tT, table, idx)

    posq3 = posq.reshape(_NW * _NREG * 2, 128)
    out_full = pl.kernel(
        _scatter_body,
        out_type=jax.ShapeDtypeStruct((_B + 128, _D), jnp.float32),
        mesh=mesh,
        scratch_types=[
            pltpu.VMEM((_CSLOT, _D), jnp.float32),
            pltpu.VMEM((2, 128), jnp.int32),
            pltpu.SemaphoreType.DMA,
        ],
        compiler_params=pltpu.CompilerParams(use_tc_tiling_on_sc=False),
    )(inter, posq3)
    return out_full[:_B]


# speedup vs baseline: 7.1651x; 7.1651x over previous
import jax
import jax.numpy as jnp
from jax import lax
from jax.experimental import pallas as pl
from jax.experimental.pallas import tpu as pltpu
from jax.experimental.pallas import tpu_sc as plsc

_NE = 1000000
_D = 64
_B = 16384

_NC = 2
_NS = 16
_NW = _NC * _NS
_EPW = 31232
_EC = 4096
_NCHUNK = 8
_TAILBASE = 999936
_ECLAMP = _TAILBASE - _EC
_MSLOT = 1024
_CSLOT = 128
_NREG = _NCHUNK + 1


def _lane(v16, s):
    return lax.squeeze(lax.slice(v16, (s,), (s + 1,)), (0,))


def _scan_body(tT, tail_t, idx_hbm, inter_out, posq_out, idxbuf, pbuf, ment,
               mpos, cent, cpos, outchunk, sem):
    wid = lax.axis_index("s") * _NC + lax.axis_index("c")
    ws = wid * _EPW
    we = jnp.where(wid == _NW - 1, _NE, ws + _EPW)
    iota = lax.iota(jnp.int32, 16)

    def pcpos(i, c):
        cpos[pl.ds(i * 16, 16)] = _B + i * 16 + iota
        return c

    neg = jnp.full((16,), -1, jnp.int32)

    def pment(i, c):
        ment[pl.ds(i * 16, 16)] = neg
        return c

    lax.fori_loop(0, _MSLOT // 16, pment, 0)

    mc = jnp.int32(0)
    for piece in range(4):
        pltpu.sync_copy(idx_hbm.at[pl.ds(piece * 4096, 4096)], idxbuf)

        def mbody(g, mc, piece=piece):
            v = idxbuf[pl.ds(g * 16, 16)]
            p16 = piece * 4096 + g * 16 + iota
            m = (v >= ws) & (v < we)
            mcc = jnp.minimum(mc, _MSLOT - 16)
            plsc.store_compressed(ment.at[pl.ds(mcc, 16)], v, mask=m)
            plsc.store_compressed(mpos.at[pl.ds(mcc, 16)], p16, mask=m)
            return mc + _lane(plsc.all_reduce_population_count(m), 0)

        mc = lax.fori_loop(0, 256, mbody, mc)

    def chunk_e0(k):
        return pl.multiple_of(jnp.minimum(ws + k * _EC, _ECLAMP), 128)

    def fire(k, p, b):
        e0 = chunk_e0(k)
        pltpu.async_copy(
            tT.at[pl.ds(p * 8, 8), pl.ds(e0, _EC)], pbuf.at[b], sem
        )

    fire(0, 0, 0)
    for k in range(_NCHUNK):
        e0 = chunk_e0(k)
        lo = ws + k * _EC
        hi = jnp.minimum(jnp.minimum(lo + _EC, we), e0 + _EC)

        lax.fori_loop(0, _CSLOT // 16, pcpos, 0)

        def cbody(g, cc):
            e16 = ment[pl.ds(g * 16, 16)]
            q16 = mpos[pl.ds(g * 16, 16)]
            m = (e16 >= lo) & (e16 < hi)
            ccc = jnp.minimum(cc, _CSLOT - 16)
            plsc.store_compressed(cent.at[pl.ds(ccc, 16)], e16, mask=m)
            plsc.store_compressed(cpos.at[pl.ds(ccc, 16)], q16, mask=m)
            return cc + _lane(plsc.all_reduce_population_count(m), 0)

        cc = lax.fori_loop(0, _MSLOT // 16, cbody, jnp.int32(0))

        for p in range(8):
            b = (k * 8 + p) % 2
            pltpu.make_async_copy(
                tT.at[pl.ds(0, 8), pl.ds(0, _EC)], pbuf.at[b], sem
            ).wait()
            if p < 7:
                fire(k, p + 1, 1 - b)
            elif k + 1 < _NCHUNK:
                fire(k + 1, 0, 1 - b)
            def ebody(j, c, b=b, p=p, cc=cc, e0=e0):
                slot = j * 16 + iota
                sm = slot < cc
                colv = jnp.clip(cent[pl.ds(j * 16, 16)] - e0, 0, _EC - 1)
                for f in range(8):
                    fv = jnp.full((16,), f, jnp.int32)
                    val = plsc.load_gather(pbuf.at[b], [fv, colv], mask=sm)
                    cv = jnp.full((16,), p * 8 + f, jnp.int32)
                    plsc.store_scatter(outchunk, [slot, cv], val, mask=sm)
                return c

            lax.fori_loop(0, _CSLOT // 16, ebody, 0)

        pltpu.sync_copy(outchunk, inter_out.at[wid * _NREG + k])
        pltpu.sync_copy(cpos, posq_out.at[pl.ds((wid * _NREG + k) * _CSLOT, _CSLOT)])

    lax.fori_loop(0, _CSLOT // 16, pcpos, 0)

    def tbody(g, cc):
        e16 = ment[pl.ds(g * 16, 16)]
        q16 = mpos[pl.ds(g * 16, 16)]
        m = e16 >= _TAILBASE
        ccc = jnp.minimum(cc, _CSLOT - 16)
        plsc.store_compressed(cent.at[pl.ds(ccc, 16)], e16, mask=m)
        plsc.store_compressed(cpos.at[pl.ds(ccc, 16)], q16, mask=m)
        return cc + _lane(plsc.all_reduce_population_count(m), 0)

    cc_t = lax.fori_loop(0, _MSLOT // 16, tbody, jnp.int32(0))
    cent0 = cent[pl.ds(0, 16)]
    for s in range(16):
        es = _lane(cent0, s)

        @pl.when(s < cc_t)
        def _():
            pltpu.sync_copy(
                tail_t.at[pl.ds(es - _TAILBASE, 1)], outchunk.at[pl.ds(s, 1)]
            )

    pltpu.sync_copy(outchunk, inter_out.at[wid * _NREG + _NCHUNK])
    pltpu.sync_copy(cpos, posq_out.at[pl.ds((wid * _NREG + _NCHUNK) * _CSLOT, _CSLOT)])


def _scatter_body(inter, posq, out_hbm, rows_v, pos_v, sem):
    wid = lax.axis_index("s") * _NC + lax.axis_index("c")
    for r in range(_NREG):
        pltpu.sync_copy(inter.at[wid * _NREG + r], rows_v)
        pltpu.sync_copy(posq.at[pl.ds(wid * _NREG + r, 1)], pos_v)
        pltpu.async_copy(rows_v, out_hbm.at[pos_v.at[0]], sem).wait()


def kernel(entities, table):
    idx = entities.astype(jnp.int32)
    tT = table.T
    tail_t = lax.slice(table, (_TAILBASE, 0), (_NE, _D))
    mesh = plsc.VectorSubcoreMesh(core_axis_name="c", subcore_axis_name="s")
    inter, posq = pl.kernel(
        _scan_body,
        out_type=(
            jax.ShapeDtypeStruct((_NW * _NREG, _CSLOT, _D), jnp.float32),
            jax.ShapeDtypeStruct((_NW * _NREG * _CSLOT,), jnp.int32),
        ),
        mesh=mesh,
        scratch_types=[
            pltpu.VMEM((4096,), jnp.int32),
            pltpu.VMEM((2, 8, _EC), jnp.float32),
            pltpu.VMEM((_MSLOT,), jnp.int32),
            pltpu.VMEM((_MSLOT,), jnp.int32),
            pltpu.VMEM((_CSLOT,), jnp.int32),
            pltpu.VMEM((_CSLOT,), jnp.int32),
            pltpu.VMEM((_CSLOT, _D), jnp.float32),
            pltpu.SemaphoreType.DMA,
        ],
        compiler_params=pltpu.CompilerParams(
            use_tc_tiling_on_sc=True, needs_layout_passes=False
        ),
    )(tT, tail_t, idx)

    posq3 = posq.reshape(_NW * _NREG, 128)
    out_full = pl.kernel(
        _scatter_body,
        out_type=jax.ShapeDtypeStruct((_B + _CSLOT, _D), jnp.float32),
        mesh=mesh,
        scratch_types=[
            pltpu.VMEM((_CSLOT, _D), jnp.float32),
            pltpu.VMEM((1, 128), jnp.int32),
            pltpu.SemaphoreType.DMA,
        ],
        compiler_params=pltpu.CompilerParams(use_tc_tiling_on_sc=False),
    )(inter, posq3)
    return out_full[:_B]


# speedup vs baseline: 7.3118x; 1.0205x over previous
import jax
import jax.numpy as jnp
from jax import lax
from jax.experimental import pallas as pl
from jax.experimental.pallas import tpu as pltpu
from jax.experimental.pallas import tpu_sc as plsc

_NE = 1000000
_D = 64
_B = 16384

_NC = 2
_NS = 16
_NW = _NC * _NS
_EPW = 31232
_EC = 4096
_NCHUNK = 8
_TAILBASE = 999936
_ECLAMP = _TAILBASE - _EC
_MSLOT = 1024
_CSLOT = 128
_NREG = _NCHUNK + 1


def _lane(v16, s):
    return lax.squeeze(lax.slice(v16, (s,), (s + 1,)), (0,))


def _scan_body(tT, tail_t, idx_hbm, inter_out, posq_out, idxbuf, pbuf, ment,
               mpos, cent, cpos, outchunk, sems):
    wid = lax.axis_index("s") * _NC + lax.axis_index("c")
    ws = wid * _EPW
    we = jnp.where(wid == _NW - 1, _NE, ws + _EPW)
    iota = lax.iota(jnp.int32, 16)

    def pcpos(i, c):
        cpos[0, pl.ds(i * 16, 16)] = _B + i * 16 + iota
        return c

    neg = jnp.full((16,), -1, jnp.int32)

    def pment(i, c):
        ment[pl.ds(i * 16, 16)] = neg
        return c

    lax.fori_loop(0, _MSLOT // 16, pment, 0)

    mc = jnp.int32(0)
    for piece in range(4):
        pltpu.sync_copy(idx_hbm.at[pl.ds(piece * 4096, 4096)], idxbuf)

        def mbody(g, mc, piece=piece):
            v = idxbuf[pl.ds(g * 16, 16)]
            p16 = piece * 4096 + g * 16 + iota
            m = (v >= ws) & (v < we)
            mcc = jnp.minimum(mc, _MSLOT - 16)
            plsc.store_compressed(ment.at[pl.ds(mcc, 16)], v, mask=m)
            plsc.store_compressed(mpos.at[pl.ds(mcc, 16)], p16, mask=m)
            return mc + _lane(plsc.all_reduce_population_count(m), 0)

        mc = lax.fori_loop(0, 256, mbody, mc)

    def chunk_e0(k):
        return pl.multiple_of(jnp.minimum(ws + k * _EC, _ECLAMP), 128)

    def fire(k, p, b):
        e0 = chunk_e0(k)
        pltpu.async_copy(
            tT.at[pl.ds(p * 8, 8), pl.ds(e0, _EC)], pbuf.at[b], sems.at[b]
        )

    fire(0, 0, 0)
    for k in range(_NCHUNK):
        e0 = chunk_e0(k)
        lo = ws + k * _EC
        hi = jnp.minimum(jnp.minimum(lo + _EC, we), e0 + _EC)

        lax.fori_loop(0, _CSLOT // 16, pcpos, 0)

        def cbody(g, cc):
            e16 = ment[pl.ds(g * 16, 16)]
            q16 = mpos[pl.ds(g * 16, 16)]
            m = (e16 >= lo) & (e16 < hi)
            ccc = jnp.minimum(cc, _CSLOT - 16)
            plsc.store_compressed(cent.at[pl.ds(ccc, 16)], e16, mask=m)
            plsc.store_compressed(cpos.at[0].at[pl.ds(ccc, 16)], q16, mask=m)
            return cc + _lane(plsc.all_reduce_population_count(m), 0)

        cc = lax.fori_loop(0, _MSLOT // 16, cbody, jnp.int32(0))

        for p in range(8):
            b = (k * 8 + p) % 2
            if p < 7:
                fire(k, p + 1, 1 - b)
            elif k + 1 < _NCHUNK:
                fire(k + 1, 0, 1 - b)
            pltpu.make_async_copy(
                tT.at[pl.ds(0, 8), pl.ds(0, _EC)], pbuf.at[b], sems.at[b]
            ).wait()
            def ebody(j, c, b=b, p=p, cc=cc, e0=e0):
                slot = j * 16 + iota
                sm = slot < cc
                colv = jnp.clip(cent[pl.ds(j * 16, 16)] - e0, 0, _EC - 1)
                for f in range(8):
                    fv = jnp.full((16,), f, jnp.int32)
                    val = plsc.load_gather(pbuf.at[b], [fv, colv], mask=sm)
                    cv = jnp.full((16,), p * 8 + f, jnp.int32)
                    plsc.store_scatter(outchunk, [slot, cv], val, mask=sm)
                return c

            lax.fori_loop(0, _CSLOT // 16, ebody, 0)

        pltpu.sync_copy(outchunk, inter_out.at[wid * _NREG + k])
        pltpu.sync_copy(cpos, posq_out.at[pl.ds(wid * _NREG + k, 1)])

    lax.fori_loop(0, _CSLOT // 16, pcpos, 0)

    def tbody(g, cc):
        e16 = ment[pl.ds(g * 16, 16)]
        q16 = mpos[pl.ds(g * 16, 16)]
        m = e16 >= _TAILBASE
        ccc = jnp.minimum(cc, _CSLOT - 16)
        plsc.store_compressed(cent.at[pl.ds(ccc, 16)], e16, mask=m)
        plsc.store_compressed(cpos.at[0].at[pl.ds(ccc, 16)], q16, mask=m)
        return cc + _lane(plsc.all_reduce_population_count(m), 0)

    cc_t = lax.fori_loop(0, _MSLOT // 16, tbody, jnp.int32(0))
    cent0 = cent[pl.ds(0, 16)]
    for s in range(16):
        es = _lane(cent0, s)

        @pl.when(s < cc_t)
        def _():
            pltpu.sync_copy(
                tail_t.at[pl.ds(es - _TAILBASE, 1)], outchunk.at[pl.ds(s, 1)]
            )

    pltpu.sync_copy(outchunk, inter_out.at[wid * _NREG + _NCHUNK])
    pltpu.sync_copy(cpos, posq_out.at[pl.ds(wid * _NREG + _NCHUNK, 1)])


def _scatter_body(inter, posq, out_hbm, rows_v, pos_v, sem):
    wid = lax.axis_index("s") * _NC + lax.axis_index("c")
    for r in range(_NREG):
        pltpu.sync_copy(inter.at[wid * _NREG + r], rows_v)
        pltpu.sync_copy(posq.at[pl.ds(wid * _NREG + r, 1)], pos_v)
        pltpu.async_copy(rows_v, out_hbm.at[pos_v.at[0]], sem).wait()


def kernel(entities, table):
    idx = entities.astype(jnp.int32)
    tT = table.T
    tail_t = lax.slice(table, (_TAILBASE, 0), (_NE, _D))
    mesh = plsc.VectorSubcoreMesh(core_axis_name="c", subcore_axis_name="s")
    inter, posq = pl.kernel(
        _scan_body,
        out_type=(
            jax.ShapeDtypeStruct((_NW * _NREG, _CSLOT, _D), jnp.float32),
            jax.ShapeDtypeStruct((_NW * _NREG, _CSLOT), jnp.int32),
        ),
        mesh=mesh,
        scratch_types=[
            pltpu.VMEM((4096,), jnp.int32),
            pltpu.VMEM((2, 8, _EC), jnp.float32),
            pltpu.VMEM((_MSLOT,), jnp.int32),
            pltpu.VMEM((_MSLOT,), jnp.int32),
            pltpu.VMEM((_CSLOT,), jnp.int32),
            pltpu.VMEM((1, _CSLOT), jnp.int32),
            pltpu.VMEM((_CSLOT, _D), jnp.float32),
            pltpu.SemaphoreType.DMA((2,)),
        ],
        compiler_params=pltpu.CompilerParams(
            use_tc_tiling_on_sc=True, needs_layout_passes=False
        ),
    )(tT, tail_t, idx)

    out_full = pl.kernel(
        _scatter_body,
        out_type=jax.ShapeDtypeStruct((_B + _CSLOT, _D), jnp.float32),
        mesh=mesh,
        scratch_types=[
            pltpu.VMEM((_CSLOT, _D), jnp.float32),
            pltpu.VMEM((1, 128), jnp.int32),
            pltpu.SemaphoreType.DMA,
        ],
        compiler_params=pltpu.CompilerParams(use_tc_tiling_on_sc=False),
    )(inter, posq)
    return out_full[:_B]
